# Initial kernel scaffold; baseline (speedup 1.0000x reference)
import jax
import jax.numpy as jnp
from jax.experimental import pallas as pl


def kernel(x, gate_w, w1, w2, w3):
    raise NotImplementedError("write your pallas kernel here")


# speedup vs baseline: 10.2436x; 10.2436x over previous
import jax
import jax.numpy as jnp
from jax import lax
from jax.experimental import pallas as pl
from jax.experimental.pallas import tpu as pltpu
from jax.experimental.pallas import tpu_sc as plsc

N = 8192
D = 768
F = 1024
E = 64
CAP = 160
EP = 168
TRASH = CAP
R = E * EP

RB = 1024
G = N // RB

NC, NS = 2, 16
NW = NC * NS
CPW = N // NW
CHUNK = 64
NCH = CPW // CHUNK


def _router_body(x_ref, gw_ref, slot_ref, aux_ref, cnt_ref, psum_ref):
    i = pl.program_id(0)

    @pl.when(i == 0)
    def _():
        cnt_ref[...] = jnp.zeros_like(cnt_ref)
        psum_ref[...] = jnp.zeros_like(psum_ref)

    xb = x_ref[...]
    gw = gw_ref[...]
    logits = lax.dot_general(xb, gw, (((1,), (1,)), ((), ())),
                             preferred_element_type=jnp.float32)
    m = jnp.max(logits, axis=1, keepdims=True)
    ex = jnp.exp(logits - m)
    probs = ex / jnp.sum(ex, axis=1, keepdims=True)

    lane = lax.broadcasted_iota(jnp.int32, (RB, E), 1)
    maxp = jnp.max(probs, axis=1, keepdims=True)
    eidx = jnp.min(jnp.where(probs >= maxp, lane, E), axis=1)
    onehot = (lane == eidx[:, None]).astype(jnp.float32)

    rr = lax.broadcasted_iota(jnp.int32, (RB, RB), 0)
    cc = lax.broadcasted_iota(jnp.int32, (RB, RB), 1)
    tril = (rr >= cc).astype(jnp.float32)
    csum = lax.dot_general(tril, onehot, (((1,), (0,)), ((), ())),
                           preferred_element_type=jnp.float32)

    carry = cnt_ref[...]
    rank = jnp.sum(onehot * (csum - 1.0 + carry), axis=1).astype(jnp.int32)
    slot = jnp.where(rank < CAP, eidx * EP + rank, TRASH)
    slot_ref[0, 0, :] = slot

    cnt_ref[...] = carry + jnp.sum(onehot, axis=0, keepdims=True)
    psum_ref[...] = psum_ref[...] + jnp.sum(probs, axis=0, keepdims=True)

    @pl.when(i == pl.num_programs(0) - 1)
    def _():
        n_tok = jnp.float32(N)
        f = cnt_ref[...] / n_tok
        p = psum_ref[...] / n_tok
        aux_ref[...] = jnp.reshape(jnp.float32(E) * jnp.sum(f * p), (1, 1))


_router = pl.pallas_call(
    _router_body,
    grid=(G,),
    in_specs=[
        pl.BlockSpec((RB, D), lambda i: (i, 0)),
        pl.BlockSpec((E, D), lambda i: (0, 0)),
    ],
    out_specs=[
        pl.BlockSpec((1, 1, RB), lambda i: (i, 0, 0)),
        pl.BlockSpec((1, 1), lambda i: (0, 0)),
    ],
    out_shape=[
        jax.ShapeDtypeStruct((G, 1, RB), jnp.int32),
        jax.ShapeDtypeStruct((1, 1), jnp.float32),
    ],
    scratch_shapes=[
        pltpu.VMEM((1, E), jnp.float32),
        pltpu.VMEM((1, E), jnp.float32),
    ],
)


def _ffn_body(buf_ref, w1_ref, w3_ref, w2_ref, out_ref):
    b = buf_ref[0]
    g = lax.dot_general(b, w1_ref[0], (((1,), (1,)), ((), ())),
                        preferred_element_type=jnp.float32)
    u = lax.dot_general(b, w3_ref[0], (((1,), (1,)), ((), ())),
                        preferred_element_type=jnp.float32)
    h = g * lax.logistic(g) * u
    o = lax.dot_general(h, w2_ref[0], (((1,), (1,)), ((), ())),
                        preferred_element_type=jnp.float32)
    rid = lax.broadcasted_iota(jnp.int32, (EP, D), 0)
    out_ref[0] = jnp.where(rid < CAP, o, 0.0)


_ffn = pl.pallas_call(
    _ffn_body,
    grid=(E,),
    in_specs=[
        pl.BlockSpec((1, EP, D), lambda i: (i, 0, 0)),
        pl.BlockSpec((1, F, D), lambda i: (i, 0, 0)),
        pl.BlockSpec((1, F, D), lambda i: (i, 0, 0)),
        pl.BlockSpec((1, D, F), lambda i: (i, 0, 0)),
    ],
    out_specs=pl.BlockSpec((1, EP, D), lambda i: (i, 0, 0)),
    out_shape=jax.ShapeDtypeStruct((E, EP, D), jnp.float32),
)


def _dispatch_body(x_hbm, slot_hbm, buf_hbm, idx_v, rows_v, sem):
    wid = lax.axis_index("s") * NC + lax.axis_index("c")
    pltpu.sync_copy(slot_hbm.at[wid], idx_v)
    for j in range(NCH):
        base = wid * CPW + j * CHUNK
        pltpu.sync_copy(x_hbm.at[pl.ds(base, CHUNK)], rows_v)
        pltpu.async_copy(rows_v, buf_hbm.at[idx_v.at[j]], sem).wait()


def _combine_body(eout_hbm, slot_hbm, out_hbm, idx_v, rows_v, sem):
    wid = lax.axis_index("s") * NC + lax.axis_index("c")
    pltpu.sync_copy(slot_hbm.at[wid], idx_v)
    for j in range(NCH):
        base = wid * CPW + j * CHUNK
        pltpu.async_copy(eout_hbm.at[idx_v.at[j]], rows_v, sem).wait()
        pltpu.sync_copy(rows_v, out_hbm.at[pl.ds(base, CHUNK)])


_SC_CACHE = {}


def _sc_kernels():
    if "k" not in _SC_CACHE:
        mesh = plsc.VectorSubcoreMesh(
            core_axis_name="c", subcore_axis_name="s",
            num_cores=NC, num_subcores=NS)
        scratch = [
            pltpu.VMEM((NCH, CHUNK), jnp.int32),
            pltpu.VMEM((CHUNK, D), jnp.float32),
            pltpu.SemaphoreType.DMA,
        ]
        dispatch = pl.kernel(
            _dispatch_body,
            out_type=jax.ShapeDtypeStruct((R, D), jnp.float32),
            mesh=mesh, scratch_types=scratch)
        combine = pl.kernel(
            _combine_body,
            out_type=jax.ShapeDtypeStruct((N, D), jnp.float32),
            mesh=mesh, scratch_types=scratch)
        _SC_CACHE["k"] = (dispatch, combine)
    return _SC_CACHE["k"]


def kernel(x, gate_w, w1, w2, w3):
    batch, seq, _ = x.shape
    xf = x.reshape(N, D)
    slot3, aux = _router(xf, gate_w)
    slot = slot3.reshape(NW, NCH, CHUNK)
    dispatch, combine = _sc_kernels()
    buf = dispatch(xf, slot)
    eout = _ffn(buf.reshape(E, EP, D), w1, w3, w2)
    out = combine(eout.reshape(R, D), slot)
    return out.reshape(batch, seq, D), aux.reshape(())


# speedup vs baseline: 10.3155x; 1.0070x over previous
import jax
import jax.numpy as jnp
from jax import lax
from jax.experimental import pallas as pl
from jax.experimental.pallas import tpu as pltpu
from jax.experimental.pallas import tpu_sc as plsc

N = 8192
D = 768
F = 1024
E = 64
CAP = 160
EP = 168
TRASH = CAP
R = E * EP

RB = 1024
G = N // RB

NC, NS = 2, 16
NW = NC * NS
CPW = N // NW
CHUNK = 64
NCH = CPW // CHUNK


def _router_body(x_ref, gw_ref, slot_ref, aux_ref, cnt_ref, psum_ref):
    i = pl.program_id(0)

    @pl.when(i == 0)
    def _():
        cnt_ref[...] = jnp.zeros_like(cnt_ref)
        psum_ref[...] = jnp.zeros_like(psum_ref)

    xb = x_ref[...]
    gw = gw_ref[...]
    logits = lax.dot_general(xb, gw, (((1,), (1,)), ((), ())),
                             preferred_element_type=jnp.float32)
    m = jnp.max(logits, axis=1, keepdims=True)
    ex = jnp.exp(logits - m)
    probs = ex / jnp.sum(ex, axis=1, keepdims=True)

    lane = lax.broadcasted_iota(jnp.int32, (RB, E), 1)
    maxp = jnp.max(probs, axis=1, keepdims=True)
    eidx = jnp.min(jnp.where(probs >= maxp, lane, E), axis=1)
    onehot = (lane == eidx[:, None]).astype(jnp.float32)

    rr = lax.broadcasted_iota(jnp.int32, (RB, RB), 0)
    cc = lax.broadcasted_iota(jnp.int32, (RB, RB), 1)
    tril = (rr >= cc).astype(jnp.float32)
    csum = lax.dot_general(tril, onehot, (((1,), (0,)), ((), ())),
                           preferred_element_type=jnp.float32)

    carry = cnt_ref[...]
    rank = jnp.sum(onehot * (csum - 1.0 + carry), axis=1).astype(jnp.int32)
    slot = jnp.where(rank < CAP, eidx * EP + rank, TRASH)
    slot_ref[0, 0, :] = slot

    cnt_ref[...] = carry + jnp.sum(onehot, axis=0, keepdims=True)
    psum_ref[...] = psum_ref[...] + jnp.sum(probs, axis=0, keepdims=True)

    @pl.when(i == pl.num_programs(0) - 1)
    def _():
        n_tok = jnp.float32(N)
        f = cnt_ref[...] / n_tok
        p = psum_ref[...] / n_tok
        aux_ref[...] = jnp.reshape(jnp.float32(E) * jnp.sum(f * p), (1, 1))


_router = pl.pallas_call(
    _router_body,
    grid=(G,),
    in_specs=[
        pl.BlockSpec((RB, D), lambda i: (i, 0)),
        pl.BlockSpec((E, D), lambda i: (0, 0)),
    ],
    out_specs=[
        pl.BlockSpec((1, 1, RB), lambda i: (i, 0, 0)),
        pl.BlockSpec((1, 1), lambda i: (0, 0)),
    ],
    out_shape=[
        jax.ShapeDtypeStruct((G, 1, RB), jnp.int32),
        jax.ShapeDtypeStruct((1, 1), jnp.float32),
    ],
    scratch_shapes=[
        pltpu.VMEM((1, E), jnp.float32),
        pltpu.VMEM((1, E), jnp.float32),
    ],
)


def _ffn_body(buf_ref, w1_ref, w3_ref, w2_ref, out_ref):
    b = buf_ref[0]
    g = lax.dot_general(b, w1_ref[0], (((1,), (1,)), ((), ())),
                        preferred_element_type=jnp.float32)
    u = lax.dot_general(b, w3_ref[0], (((1,), (1,)), ((), ())),
                        preferred_element_type=jnp.float32)
    h = g * lax.logistic(g) * u
    o = lax.dot_general(h, w2_ref[0], (((1,), (1,)), ((), ())),
                        preferred_element_type=jnp.float32)
    rid = lax.broadcasted_iota(jnp.int32, (EP, D), 0)
    out_ref[0] = jnp.where(rid < CAP, o, 0.0)


_ffn = pl.pallas_call(
    _ffn_body,
    grid=(E,),
    in_specs=[
        pl.BlockSpec((1, EP, D), lambda i: (i, 0, 0)),
        pl.BlockSpec((1, F, D), lambda i: (i, 0, 0)),
        pl.BlockSpec((1, F, D), lambda i: (i, 0, 0)),
        pl.BlockSpec((1, D, F), lambda i: (i, 0, 0)),
    ],
    out_specs=pl.BlockSpec((1, EP, D), lambda i: (i, 0, 0)),
    out_shape=jax.ShapeDtypeStruct((E, EP, D), jnp.float32),
)


def _dispatch_body(x_hbm, slot_hbm, buf_hbm, idx_v, rows_v, sems):
    wid = lax.axis_index("s") * NC + lax.axis_index("c")
    pltpu.sync_copy(slot_hbm.at[wid], idx_v)

    def load(j):
        base = wid * CPW + j * CHUNK
        return pltpu.async_copy(
            x_hbm.at[pl.ds(base, CHUNK)], rows_v.at[j % 2], sems.at[j % 2])

    loads = [load(0)]
    for j in range(NCH):
        if j + 1 < NCH:
            loads.append(load(j + 1))
        loads[j].wait()
        pltpu.async_copy(rows_v.at[j % 2], buf_hbm.at[idx_v.at[j]],
                         sems.at[2]).wait()


def _combine_body(eout_hbm, slot_hbm, out_hbm, idx_v, rows_v, sems):
    wid = lax.axis_index("s") * NC + lax.axis_index("c")
    pltpu.sync_copy(slot_hbm.at[wid], idx_v)

    def gather(j):
        return pltpu.async_copy(
            eout_hbm.at[idx_v.at[j]], rows_v.at[j % 2], sems.at[j % 2])

    gathers = [gather(0)]
    for j in range(NCH):
        if j + 1 < NCH:
            gathers.append(gather(j + 1))
        gathers[j].wait()
        base = wid * CPW + j * CHUNK
        pltpu.async_copy(rows_v.at[j % 2], out_hbm.at[pl.ds(base, CHUNK)],
                         sems.at[2]).wait()


_SC_CACHE = {}


def _sc_kernels():
    if "k" not in _SC_CACHE:
        mesh = plsc.VectorSubcoreMesh(
            core_axis_name="c", subcore_axis_name="s",
            num_cores=NC, num_subcores=NS)
        scratch = [
            pltpu.VMEM((NCH, CHUNK), jnp.int32),
            pltpu.VMEM((2, CHUNK, D), jnp.float32),
            pltpu.SemaphoreType.DMA((3,)),
        ]
        dispatch = pl.kernel(
            _dispatch_body,
            out_type=jax.ShapeDtypeStruct((R, D), jnp.float32),
            mesh=mesh, scratch_types=scratch)
        combine = pl.kernel(
            _combine_body,
            out_type=jax.ShapeDtypeStruct((N, D), jnp.float32),
            mesh=mesh, scratch_types=scratch)
        _SC_CACHE["k"] = (dispatch, combine)
    return _SC_CACHE["k"]


def kernel(x, gate_w, w1, w2, w3):
    batch, seq, _ = x.shape
    xf = x.reshape(N, D)
    slot3, aux = _router(xf, gate_w)
    slot = slot3.reshape(NW, NCH, CHUNK)
    dispatch, combine = _sc_kernels()
    buf = dispatch(xf, slot)
    eout = _ffn(buf.reshape(E, EP, D), w1, w3, w2)
    out = combine(eout.reshape(R, D), slot)
    return out.reshape(batch, seq, D), aux.reshape(())
